# baseline (speedup 1.0000x reference)
import functools

import jax
import jax.numpy as jnp
from jax import lax
from jax.experimental import pallas as pl
from jax.experimental.pallas import tpu as pltpu
from jax.experimental.pallas import tpu_sc as plsc

_V, _D, _B = 1000000, 32, 16384
_NC, _NS = 2, 16
_NW = _NC * _NS
_C = 1024
_NSLAB_UNIFORM = 30
_N_EXTRA = 976 - 32 * _NSLAB_UNIFORM
_T1_OFF = 976 * _C
_T2_OFF = _T1_OFF + 512


def _worker_id():
  return lax.axis_index("s") * _NC + lax.axis_index("c")


@functools.cache
def _build_detile():
  mesh = plsc.VectorSubcoreMesh(core_axis_name="c", subcore_axis_name="s")
  bufw = _D * _C

  @functools.partial(
      pl.kernel,
      out_type=[jax.ShapeDtypeStruct((_D * _V,), jnp.float32)] * 3,
      mesh=mesh,
      compiler_params=pltpu.CompilerParams(use_tc_tiling_on_sc=True),
      scratch_types=[
          pltpu.VMEM((bufw,), jnp.float32),
          pltpu.VMEM((bufw,), jnp.float32),
          pltpu.SemaphoreType.DMA,
          pltpu.SemaphoreType.DMA,
          pltpu.SemaphoreType.DMA,
          pltpu.SemaphoreType.DMA,
      ],
  )
  def detile(t1, t2, t3, tail, s1, s2, s3,
             bufa, bufb, in_a, in_b, out_a, out_b):
    wid = _worker_id()
    bufs = (bufa, bufb)
    in_sems = (in_a, in_b)
    out_sems = (out_a, out_b)

    def fire_in(src, col, width, buf, sem):
      for c in range(_D):
        pltpu.async_copy(src.at[c, pl.ds(col, width)],
                         buf.at[pl.ds(c * _C, width)], sem)

    def fire_out(dst, col, width, buf, sem):
      for c in range(_D):
        pltpu.async_copy(buf.at[pl.ds(c * _C, width)],
                         dst.at[pl.ds(c * _V + col, width)], sem)

    def drain(src, words, sem):
      pltpu.make_async_copy(src.at[0, pl.ds(0, words)],
                            bufa.at[pl.ds(0, words)], sem).wait()

    for src, dst in ((t1, s1), (t2, s2), (t3, s3)):
      def slab(k, _, src=src, dst=dst):
        col = (wid * _NSLAB_UNIFORM + k) * _C
        for par in (0, 1):
          @pl.when(k % 2 == par)
          def _(par=par):
            buf = bufs[par]

            @pl.when(k >= 2)
            def _():
              drain(src, _D * _C, out_sems[par])
            fire_in(src, col, _C, buf, in_sems[par])
            drain(src, _D * _C, in_sems[par])
            fire_out(dst, col, _C, buf, out_sems[par])
        return 0

      lax.fori_loop(0, _NSLAB_UNIFORM, slab, 0)
      drain(src, _D * _C, out_sems[_NSLAB_UNIFORM % 2])
      drain(src, _D * _C, out_sems[(_NSLAB_UNIFORM + 1) % 2])

      @pl.when(wid < _N_EXTRA)
      def _(src=src, dst=dst):
        col = (960 + wid) * _C
        fire_in(src, col, _C, bufa, in_a)
        drain(src, _D * _C, in_a)
        fire_out(dst, col, _C, bufa, out_a)
        drain(src, _D * _C, out_a)

      @pl.when(wid == 30)
      def _(src=src, dst=dst):
        fire_in(src, _T1_OFF, 512, bufa, in_a)
        drain(src, _D * 512, in_a)
        fire_out(dst, _T1_OFF, 512, bufa, out_a)
        drain(src, _D * 512, out_a)

    @pl.when(wid == 31)
    def _():
      pltpu.sync_copy(tail, bufb.at[pl.ds(0, 3 * _D * 64)])
      for i, dst in enumerate((s1, s2, s3)):
        for c in range(_D):
          pltpu.async_copy(bufb.at[pl.ds((i * _D + c) * 64, 64)],
                           dst.at[pl.ds(c * _V + _T2_OFF, 64)], out_b)
      drain(t1, 3 * _D * 64, out_b)

  return detile


@functools.cache
def _build_gather():
  mesh = plsc.VectorSubcoreMesh(core_axis_name="c", subcore_axis_name="s")
  nw_words = (_B // _NW) * _D

  bw = _B // _NW

  @functools.partial(
      pl.kernel,
      out_type=[jax.ShapeDtypeStruct((_B * _D,), jnp.float32)] * 4,
      mesh=mesh,
      compiler_params=pltpu.CompilerParams(use_tc_tiling_on_sc=False,
                                           needs_layout_passes=False),
      scratch_types=[
          pltpu.VMEM((bw,), jnp.int32),
          pltpu.VMEM((bw,), jnp.int32),
          pltpu.VMEM((nw_words,), jnp.int32),
          pltpu.VMEM((nw_words,), jnp.int32),
          pltpu.VMEM((nw_words,), jnp.float32),
          pltpu.VMEM((nw_words,), jnp.float32),
          pltpu.VMEM((nw_words,), jnp.float32),
          pltpu.VMEM((nw_words,), jnp.float32),
          pltpu.SemaphoreType.DMA,
      ],
  )
  def gather(s1f, s2f, s3f, vi_hbm, vj_hbm,
             o1, o2, o3, o4, iv, jv, ei, ej, r1, r2, r3, r4, sem):
    wid = _worker_id()
    base = wid * nw_words
    ibase = wid * bw
    pltpu.sync_copy(vi_hbm.at[pl.ds(ibase, bw)], iv)
    pltpu.sync_copy(vj_hbm.at[pl.ds(ibase, bw)], jv)

    lane = lax.iota(jnp.int32, 16)

    def expand(chunk, _):
      ri = iv[pl.ds(chunk * 16, 16)]
      rj = jv[pl.ds(chunk * 16, 16)]
      pos0 = (chunk * 16 + lane) * _D
      for c in range(_D):
        plsc.store_scatter(ei, [pos0 + c], ri + c * _V)
        plsc.store_scatter(ej, [pos0 + c], rj + c * _V)
      return 0

    lax.fori_loop(0, bw // 16, expand, 0)
    c1 = pltpu.async_copy(s1f.at[ei], r1, sem)
    c2 = pltpu.async_copy(s1f.at[ej], r2, sem)
    c3 = pltpu.async_copy(s2f.at[ei], r3, sem)
    c4 = pltpu.async_copy(s3f.at[ej], r4, sem)
    c1.wait()
    pltpu.sync_copy(r1, o1.at[pl.ds(base, nw_words)])
    c2.wait()
    pltpu.sync_copy(r2, o2.at[pl.ds(base, nw_words)])
    c3.wait()
    pltpu.sync_copy(r3, o3.at[pl.ds(base, nw_words)])
    c4.wait()
    pltpu.sync_copy(r4, o4.at[pl.ds(base, nw_words)])

  return gather


def kernel(nodeindex, v_i, v_j, device, embeddings, second_embeddings,
           context_embeddings):
  detile = _build_detile()
  gather = _build_gather()

  tails = jnp.concatenate(
      [t[_T2_OFF:, :].T.reshape(-1)
       for t in (embeddings, second_embeddings, context_embeddings)])
  s1f, s2f, s3f = detile(embeddings.T, second_embeddings.T,
                         context_embeddings.T, tails)

  f1, f2, f3, f4 = gather(s1f, s2f, s3f,
                          v_i.astype(jnp.int32), v_j.astype(jnp.int32))
  return (jnp.reshape(f1, (_B, _D)), jnp.reshape(f2, (_B, _D)),
          jnp.reshape(f3, (_B, _D)), jnp.reshape(f4, (_B, _D)))


# speedup vs baseline: 1.0592x; 1.0592x over previous
import functools

import jax
import jax.numpy as jnp
from jax import lax
from jax.experimental import pallas as pl
from jax.experimental.pallas import tpu as pltpu
from jax.experimental.pallas import tpu_sc as plsc

_V, _D, _B = 1000000, 32, 16384
_NC, _NS = 2, 16
_NW = _NC * _NS
_C = 768
_NSLABS = 1302
_N_HI_W = _NSLABS - 40 * _NW
_T2_OFF = _NSLABS * _C


def _worker_id():
  return lax.axis_index("s") * _NC + lax.axis_index("c")


@functools.cache
def _build_detile():
  mesh = plsc.VectorSubcoreMesh(core_axis_name="c", subcore_axis_name="s")
  bufw = _D * _C

  @functools.partial(
      pl.kernel,
      out_type=[jax.ShapeDtypeStruct((_D * _V,), jnp.float32)] * 3,
      mesh=mesh,
      compiler_params=pltpu.CompilerParams(use_tc_tiling_on_sc=True),
      scratch_types=[
          pltpu.VMEM((bufw,), jnp.float32),
          pltpu.VMEM((bufw,), jnp.float32),
          pltpu.VMEM((bufw,), jnp.float32),
          pltpu.VMEM((bufw,), jnp.float32),
          [pltpu.SemaphoreType.DMA] * 4,
          [pltpu.SemaphoreType.DMA] * 4,
      ],
  )
  def detile(t1, t2, t3, tail, s1, s2, s3,
             buf0, buf1, buf2, buf3, in_sems, out_sems):
    wid = _worker_id()
    bufs = (buf0, buf1, buf2, buf3)
    hi = wid < _N_HI_W
    nslab = jnp.where(hi, 41, 40)
    start = jnp.where(hi, wid * 41, _N_HI_W * 41 + (wid - _N_HI_W) * 40)

    def fire_in(src, col, buf, sem):
      for c in range(_D):
        pltpu.async_copy(src.at[c, pl.ds(col, _C)],
                         buf.at[pl.ds(c * _C, _C)], sem)

    def fire_out(dst, col, buf, sem):
      for c in range(_D):
        pltpu.async_copy(buf.at[pl.ds(c * _C, _C)],
                         dst.at[pl.ds(c * _V + col, _C)], sem)

    def drain(src, words, sem):
      pltpu.make_async_copy(src.at[0, pl.ds(0, words)],
                            buf0.at[pl.ds(0, words)], sem).wait()

    for src, dst in ((t1, s1), (t2, s2), (t3, s3)):
      fire_in(src, start * _C, bufs[0], in_sems[0])
      fire_in(src, (start + 1) * _C, bufs[1], in_sems[1])

      def slab(k, _, src=src, dst=dst):
        col = (start + k) * _C
        for par in range(4):
          @pl.when(k % 4 == par)
          def _(par=par):
            drain(src, _D * _C, in_sems[par])
            fire_out(dst, col, bufs[par], out_sems[par])
            nxt = (par + 2) % 4

            @pl.when(k >= 2)
            def _():
              drain(src, _D * _C, out_sems[nxt])

            @pl.when(k + 2 < nslab)
            def _():
              fire_in(src, (start + k + 2) * _C, bufs[nxt], in_sems[nxt])
        return 0

      lax.fori_loop(0, nslab, slab, 0)
      @pl.when(hi)
      def _(src=src):
        drain(src, _D * _C, out_sems[0])
        drain(src, _D * _C, out_sems[3])

      @pl.when(jnp.logical_not(hi))
      def _(src=src):
        drain(src, _D * _C, out_sems[3])
        drain(src, _D * _C, out_sems[2])

    @pl.when(wid == _NW - 1)
    def _():
      pltpu.sync_copy(tail, buf1.at[pl.ds(0, 3 * _D * 64)])
      for i, dst in enumerate((s1, s2, s3)):
        for c in range(_D):
          pltpu.async_copy(buf1.at[pl.ds((i * _D + c) * 64, 64)],
                           dst.at[pl.ds(c * _V + _T2_OFF, 64)], out_sems[1])
      drain(t1, 3 * _D * 64, out_sems[1])

  return detile


@functools.cache
def _build_gather():
  mesh = plsc.VectorSubcoreMesh(core_axis_name="c", subcore_axis_name="s")
  nw_words = (_B // _NW) * _D

  @functools.partial(
      pl.kernel,
      out_type=[jax.ShapeDtypeStruct((_B * _D,), jnp.float32)] * 4,
      mesh=mesh,
      compiler_params=pltpu.CompilerParams(use_tc_tiling_on_sc=False),
      scratch_types=[
          pltpu.VMEM((nw_words,), jnp.int32),
          pltpu.VMEM((nw_words,), jnp.int32),
          pltpu.VMEM((nw_words,), jnp.float32),
          pltpu.VMEM((nw_words,), jnp.float32),
          pltpu.VMEM((nw_words,), jnp.float32),
          pltpu.VMEM((nw_words,), jnp.float32),
          pltpu.SemaphoreType.DMA,
      ],
  )
  def gather(s1f, s2f, s3f, ei_hbm, ej_hbm,
             o1, o2, o3, o4, ei, ej, r1, r2, r3, r4, sem):
    wid = _worker_id()
    base = wid * nw_words
    pltpu.sync_copy(ei_hbm.at[pl.ds(base, nw_words)], ei)
    pltpu.sync_copy(ej_hbm.at[pl.ds(base, nw_words)], ej)
    c1 = pltpu.async_copy(s1f.at[ei], r1, sem)
    c2 = pltpu.async_copy(s1f.at[ej], r2, sem)
    c3 = pltpu.async_copy(s2f.at[ei], r3, sem)
    c4 = pltpu.async_copy(s3f.at[ej], r4, sem)
    c1.wait()
    pltpu.sync_copy(r1, o1.at[pl.ds(base, nw_words)])
    c2.wait()
    pltpu.sync_copy(r2, o2.at[pl.ds(base, nw_words)])
    c3.wait()
    pltpu.sync_copy(r3, o3.at[pl.ds(base, nw_words)])
    c4.wait()
    pltpu.sync_copy(r4, o4.at[pl.ds(base, nw_words)])

  return gather


def kernel(nodeindex, v_i, v_j, device, embeddings, second_embeddings,
           context_embeddings):
  detile = _build_detile()
  gather = _build_gather()

  tails = jnp.concatenate(
      [t[_T2_OFF:, :].T.reshape(-1)
       for t in (embeddings, second_embeddings, context_embeddings)])
  s1f, s2f, s3f = detile(embeddings.T, second_embeddings.T,
                         context_embeddings.T, tails)

  coff = (jnp.arange(_D, dtype=jnp.int32) * _V)[None, :]
  ei = jnp.reshape(v_i.astype(jnp.int32)[:, None] + coff, (_B * _D,))
  ej = jnp.reshape(v_j.astype(jnp.int32)[:, None] + coff, (_B * _D,))

  f1, f2, f3, f4 = gather(s1f, s2f, s3f, ei, ej)
  return (jnp.reshape(f1, (_B, _D)), jnp.reshape(f2, (_B, _D)),
          jnp.reshape(f3, (_B, _D)), jnp.reshape(f4, (_B, _D)))


# speedup vs baseline: 1.0636x; 1.0042x over previous
import functools

import jax
import jax.numpy as jnp
from jax import lax
from jax.experimental import pallas as pl
from jax.experimental.pallas import tpu as pltpu
from jax.experimental.pallas import tpu_sc as plsc

_V, _D, _B = 1000000, 32, 16384
_NC, _NS = 2, 16
_NW = _NC * _NS
_C = 768
_NSLABS = 1302
_N_HI_W = _NSLABS - 40 * _NW
_T2_OFF = _NSLABS * _C


def _worker_id():
  return lax.axis_index("s") * _NC + lax.axis_index("c")


@functools.cache
def _build_detile():
  mesh = plsc.VectorSubcoreMesh(core_axis_name="c", subcore_axis_name="s")
  bufw = _D * _C

  @functools.partial(
      pl.kernel,
      out_type=[jax.ShapeDtypeStruct((_D * _V,), jnp.float32)] * 3,
      mesh=mesh,
      compiler_params=pltpu.CompilerParams(use_tc_tiling_on_sc=True),
      scratch_types=[
          pltpu.VMEM((bufw,), jnp.float32),
          pltpu.VMEM((bufw,), jnp.float32),
          pltpu.VMEM((bufw,), jnp.float32),
          pltpu.VMEM((bufw,), jnp.float32),
          [pltpu.SemaphoreType.DMA] * 4,
          [pltpu.SemaphoreType.DMA] * 4,
      ],
  )
  def detile(t1, t2, t3, tail, s1, s2, s3,
             buf0, buf1, buf2, buf3, in_sems, out_sems):
    wid = _worker_id()
    bufs = (buf0, buf1, buf2, buf3)
    hi = wid < _N_HI_W
    nslab = jnp.where(hi, 41, 40)
    start = jnp.where(hi, wid * 41, _N_HI_W * 41 + (wid - _N_HI_W) * 40)

    def fire_in(src, col, buf, sem):
      for c in range(_D):
        pltpu.async_copy(src.at[c, pl.ds(col, _C)],
                         buf.at[pl.ds(c * _C, _C)], sem)

    def fire_out(dst, col, buf, sem):
      for c in range(_D):
        pltpu.async_copy(buf.at[pl.ds(c * _C, _C)],
                         dst.at[pl.ds(c * _V + col, _C)], sem)

    def drain(src, words, sem):
      pltpu.make_async_copy(src.at[0, pl.ds(0, words)],
                            buf0.at[pl.ds(0, words)], sem).wait()

    for src, dst in ((t1, s1), (t2, s2), (t3, s3)):
      fire_in(src, start * _C, bufs[0], in_sems[0])
      fire_in(src, (start + 1) * _C, bufs[1], in_sems[1])

      def slab(k, _, src=src, dst=dst):
        col = (start + k) * _C
        for par in range(4):
          @pl.when(k % 4 == par)
          def _(par=par):
            drain(src, _D * _C, in_sems[par])
            fire_out(dst, col, bufs[par], out_sems[par])
            nxt = (par + 2) % 4

            @pl.when(k >= 2)
            def _():
              drain(src, _D * _C, out_sems[nxt])

            @pl.when(k + 2 < nslab)
            def _():
              fire_in(src, (start + k + 2) * _C, bufs[nxt], in_sems[nxt])
        return 0

      lax.fori_loop(0, nslab, slab, 0)
      @pl.when(hi)
      def _(src=src):
        drain(src, _D * _C, out_sems[0])
        drain(src, _D * _C, out_sems[3])

      @pl.when(jnp.logical_not(hi))
      def _(src=src):
        drain(src, _D * _C, out_sems[3])
        drain(src, _D * _C, out_sems[2])

    @pl.when(wid == _NW - 1)
    def _():
      pltpu.sync_copy(tail, buf1.at[pl.ds(0, 3 * _D * 64)])
      for i, dst in enumerate((s1, s2, s3)):
        for c in range(_D):
          pltpu.async_copy(buf1.at[pl.ds((i * _D + c) * 64, 64)],
                           dst.at[pl.ds(c * _V + _T2_OFF, 64)], out_sems[1])
      drain(t1, 3 * _D * 64, out_sems[1])

  return detile


@functools.cache
def _build_gather():
  mesh = plsc.VectorSubcoreMesh(core_axis_name="c", subcore_axis_name="s")
  nw_words = (_B // _NW) * _D

  @functools.partial(
      pl.kernel,
      out_type=[jax.ShapeDtypeStruct((_B * _D,), jnp.float32)] * 4,
      mesh=mesh,
      compiler_params=pltpu.CompilerParams(use_tc_tiling_on_sc=False),
      scratch_types=[
          pltpu.VMEM((nw_words,), jnp.int32),
          pltpu.VMEM((nw_words,), jnp.int32),
          pltpu.VMEM((nw_words,), jnp.float32),
          pltpu.VMEM((nw_words,), jnp.float32),
          pltpu.VMEM((nw_words,), jnp.float32),
          pltpu.VMEM((nw_words,), jnp.float32),
          pltpu.SemaphoreType.DMA,
          pltpu.SemaphoreType.DMA,
      ],
  )
  def gather(s1f, s2f, s3f, ei_hbm, ej_hbm,
             o1, o2, o3, o4, ei, ej, r1, r2, r3, r4, sem, osem):
    wid = _worker_id()
    base = wid * nw_words
    pltpu.sync_copy(ei_hbm.at[pl.ds(base, nw_words)], ei)
    c1 = pltpu.async_copy(s1f.at[ei], r1, sem)
    c3 = pltpu.async_copy(s2f.at[ei], r3, sem)
    pltpu.sync_copy(ej_hbm.at[pl.ds(base, nw_words)], ej)
    c2 = pltpu.async_copy(s1f.at[ej], r2, sem)
    c4 = pltpu.async_copy(s3f.at[ej], r4, sem)
    c1.wait()
    pltpu.async_copy(r1, o1.at[pl.ds(base, nw_words)], osem)
    c3.wait()
    pltpu.async_copy(r3, o3.at[pl.ds(base, nw_words)], osem)
    c2.wait()
    pltpu.async_copy(r2, o2.at[pl.ds(base, nw_words)], osem)
    c4.wait()
    pltpu.async_copy(r4, o4.at[pl.ds(base, nw_words)], osem)
    for _ in range(4):
      pltpu.make_async_copy(s1f.at[pl.ds(0, nw_words)],
                            r1, osem).wait()

  return gather


def kernel(nodeindex, v_i, v_j, device, embeddings, second_embeddings,
           context_embeddings):
  detile = _build_detile()
  gather = _build_gather()

  tails = jnp.concatenate(
      [t[_T2_OFF:, :].T.reshape(-1)
       for t in (embeddings, second_embeddings, context_embeddings)])
  s1f, s2f, s3f = detile(embeddings.T, second_embeddings.T,
                         context_embeddings.T, tails)

  coff = (jnp.arange(_D, dtype=jnp.int32) * _V)[None, :]
  ei = jnp.reshape(v_i.astype(jnp.int32)[:, None] + coff, (_B * _D,))
  ej = jnp.reshape(v_j.astype(jnp.int32)[:, None] + coff, (_B * _D,))

  f1, f2, f3, f4 = gather(s1f, s2f, s3f, ei, ej)
  return (jnp.reshape(f1, (_B, _D)), jnp.reshape(f2, (_B, _D)),
          jnp.reshape(f3, (_B, _D)), jnp.reshape(f4, (_B, _D)))
